# baseline (speedup 1.0000x reference)
import functools

import jax
import jax.numpy as jnp
from jax import lax
from jax.experimental import pallas as pl
from jax.experimental.pallas import tpu as pltpu
from jax.experimental.pallas import tpu_sc as plsc

D_MODEL = 128
ATTR_DIM = 4
NUM_PRIMITIVES = 10
LANES = 16
NJ = D_MODEL // LANES

NUM_CORES = 2
NUM_SUBCORES = 16
NUM_WORKERS = NUM_CORES * NUM_SUBCORES

PANEL = 128
CN = 40
PB = 4


N_SC_FRAC_NUM = 3
N_SC_FRAC_DEN = 5
TC_BB = 128
TC_BN = 40


def _run(ids_t, attr_t, table, wt, n_b, n_n, n_sc):
    n_chunks = n_sc // CN
    n_pieces = PANEL // PB

    mesh = plsc.VectorSubcoreMesh(core_axis_name="c", subcore_axis_name="s")

    @functools.partial(
        pl.kernel,
        mesh=mesh,
        out_type=jax.ShapeDtypeStruct((n_b * n_n, D_MODEL), jnp.float32),
        scratch_types=[
            pltpu.VMEM((NUM_PRIMITIVES, D_MODEL), jnp.float32),
            pltpu.VMEM((ATTR_DIM, D_MODEL), jnp.float32),
            pltpu.VMEM((CN + 1, PANEL), jnp.int32),
            pltpu.VMEM((CN + 1, ATTR_DIM, PANEL), jnp.float32),
            pltpu.VMEM((PB * CN, D_MODEL), jnp.float32),
            pltpu.VMEM((PB * CN, D_MODEL), jnp.float32),
            pltpu.SemaphoreType.DMA,
            pltpu.SemaphoreType.DMA,
        ],
    )
    def k(ids_hbm, attr_hbm, table_hbm, wt_hbm, out_hbm,
          table_v, wt_v, ids_v, attr_v, out0, out1, sout0, sout1):
        outb = (out0, out1)
        sout = (sout0, sout1)

        wid = lax.axis_index("s") * NUM_CORES + lax.axis_index("c")
        b0w = wid * PANEL
        pltpu.sync_copy(table_hbm, table_v)
        pltpu.sync_copy(wt_hbm, wt_v)
        w = [[wt_v[a, pl.ds(LANES * j, LANES)] for j in range(NJ)]
             for a in range(ATTR_DIM)]

        def wait_out(p):
            for bb in range(PB):
                pltpu.make_async_copy(
                    outb[p].at[pl.ds(bb * CN, CN), :],
                    out_hbm.at[pl.ds(0, CN)], sout[p]).wait()

        def chunk_body(c, carry):
            n0 = c * CN
            pltpu.sync_copy(ids_hbm.at[pl.ds(n0, CN), pl.ds(b0w, PANEL)],
                            ids_v.at[pl.ds(0, CN), :])
            pltpu.sync_copy(
                attr_hbm.at[pl.ds(n0, CN), :, pl.ds(b0w, PANEL)],
                attr_v.at[pl.ds(0, CN), :, :])

            def piece_pair(pp, carry2):
                for p in range(2):
                    pb = 2 * pp + p
                    bcol = PB * pb

                    @pl.when(c * n_pieces + pb >= 2)
                    def _():
                        wait_out(p)

                    out_v = outb[p]

                    @plsc.parallel_loop(0, CN)
                    def _(n):
                        idsg = ids_v[n, pl.ds(bcol, LANES)]
                        avg = [attr_v[n, a, pl.ds(bcol, LANES)]
                               for a in range(ATTR_DIM)]
                        for bb in range(PB):
                            pid = idsg[bb]
                            a0 = avg[0][bb]
                            a1 = avg[1][bb]
                            a2 = avg[2][bb]
                            a3 = avg[3][bb]
                            for j in range(NJ):
                                e = table_v[pid, pl.ds(LANES * j, LANES)]
                                r = e + a0 * w[0][j] + a1 * w[1][j] \
                                    + a2 * w[2][j] + a3 * w[3][j]
                                out_v[bb * CN + n, pl.ds(LANES * j, LANES)] = r

                    for bb in range(PB):
                        row0 = (b0w + bcol + bb) * n_n + n0
                        pltpu.async_copy(out_v.at[pl.ds(bb * CN, CN), :],
                                         out_hbm.at[pl.ds(row0, CN)], sout[p])
                return carry2

            lax.fori_loop(0, n_pieces // 2, piece_pair, 0)
            return carry

        lax.fori_loop(0, n_chunks, chunk_body, 0)
        wait_out(0)
        wait_out(1)

    return k(ids_t, attr_t, table, wt)


def _tc_kernel(ids_ref, attr_ref, tabp_ref, wt_ref, alias_ref, out_ref):
    t = TC_BB * TC_BN
    idsb = jnp.transpose(ids_ref[...].astype(jnp.float32))
    iota = lax.broadcasted_iota(jnp.int32,
                                (TC_BB, TC_BN, D_MODEL), 2).astype(jnp.float32)
    oh = (idsb[:, :, None] == iota).astype(jnp.float32)
    base = jnp.dot(oh.reshape(t, D_MODEL), tabp_ref[...],
                   preferred_element_type=jnp.float32)
    acc = base.reshape(TC_BB, TC_BN, D_MODEL)
    for a in range(ATTR_DIM):
        atta = jnp.transpose(attr_ref[:, a, :])
        acc = acc + atta[:, :, None] * wt_ref[a, :][None, None, :]
    out_ref[...] = acc


def _tc_fill(out3, ids_t, attr_t, tab_pad, wt, n_sc):
    N, B = ids_t.shape
    joff = n_sc // TC_BN
    grid = (B // TC_BB, (N - n_sc) // TC_BN)
    return pl.pallas_call(
        _tc_kernel,
        grid=grid,
        in_specs=[
            pl.BlockSpec((TC_BN, TC_BB), lambda i, j: (j + joff, i)),
            pl.BlockSpec((TC_BN, ATTR_DIM, TC_BB),
                         lambda i, j: (j + joff, 0, i)),
            pl.BlockSpec((D_MODEL, D_MODEL), lambda i, j: (0, 0)),
            pl.BlockSpec((ATTR_DIM, D_MODEL), lambda i, j: (0, 0)),
            pl.BlockSpec(memory_space=pl.ANY),
        ],
        out_specs=pl.BlockSpec((TC_BB, TC_BN, D_MODEL),
                               lambda i, j: (i, j + joff, 0)),
        out_shape=jax.ShapeDtypeStruct((B, N, D_MODEL), jnp.float32),
        input_output_aliases={4: 0},
    )(ids_t, attr_t, tab_pad, wt, out3)


@functools.partial(jax.jit, static_argnames=("n_b", "n_n", "n_sc"))
def _run_all(ids_t, attr_t, table, wt, n_b, n_n, n_sc):
    out = _run(ids_t, attr_t, table, wt, n_b, n_n, n_sc)
    out3 = out.reshape(n_b, n_n, D_MODEL)
    if n_sc < n_n:
        tab_pad = jnp.concatenate(
            [table, jnp.zeros((D_MODEL - NUM_PRIMITIVES, D_MODEL),
                              jnp.float32)], axis=0)
        out3 = _tc_fill(out3, ids_t, attr_t, tab_pad, wt, n_sc)
    return out3


def kernel(primitive_ids, attr_vec, emb_table, attr_W):
    B, N = primitive_ids.shape
    n_sc = (N * N_SC_FRAC_NUM // (N_SC_FRAC_DEN * CN)) * CN
    if n_sc <= 0 or (N - n_sc) % TC_BN != 0 or N % CN != 0:
        n_sc = N
    ids32 = primitive_ids.astype(jnp.int32)
    attrf = attr_vec.astype(jnp.float32)
    ids_t = ids32.T
    attr_t = jnp.transpose(attrf, (1, 2, 0))
    wt = attr_W.astype(jnp.float32).T
    out3 = _run_all(ids_t, attr_t,
                    emb_table.astype(jnp.float32), wt, B, N, n_sc)
    return out3


# speedup vs baseline: 1.0233x; 1.0233x over previous
import functools

import jax
import jax.numpy as jnp
from jax import lax
from jax.experimental import pallas as pl
from jax.experimental.pallas import tpu as pltpu
from jax.experimental.pallas import tpu_sc as plsc

D_MODEL = 128
ATTR_DIM = 4
NUM_PRIMITIVES = 10
LANES = 16
NJ = D_MODEL // LANES

NUM_CORES = 2
NUM_SUBCORES = 16
NUM_WORKERS = NUM_CORES * NUM_SUBCORES

PANEL = 128
CN = 40
PB = 4


N_SC_FRAC_NUM = 2
N_SC_FRAC_DEN = 5
TC_BB = 128
TC_BN = 40


def _run(ids_t, attr_t, table, wt, n_b, n_n, n_sc):
    n_chunks = n_sc // CN
    n_pieces = PANEL // PB

    mesh = plsc.VectorSubcoreMesh(core_axis_name="c", subcore_axis_name="s")

    @functools.partial(
        pl.kernel,
        mesh=mesh,
        out_type=jax.ShapeDtypeStruct((n_b * n_n, D_MODEL), jnp.float32),
        scratch_types=[
            pltpu.VMEM((NUM_PRIMITIVES, D_MODEL), jnp.float32),
            pltpu.VMEM((ATTR_DIM, D_MODEL), jnp.float32),
            pltpu.VMEM((CN + 1, PANEL), jnp.int32),
            pltpu.VMEM((CN + 1, ATTR_DIM, PANEL), jnp.float32),
            pltpu.VMEM((PB * CN, D_MODEL), jnp.float32),
            pltpu.VMEM((PB * CN, D_MODEL), jnp.float32),
            pltpu.SemaphoreType.DMA,
            pltpu.SemaphoreType.DMA,
        ],
    )
    def k(ids_hbm, attr_hbm, table_hbm, wt_hbm, out_hbm,
          table_v, wt_v, ids_v, attr_v, out0, out1, sout0, sout1):
        outb = (out0, out1)
        sout = (sout0, sout1)

        wid = lax.axis_index("s") * NUM_CORES + lax.axis_index("c")
        b0w = wid * PANEL
        pltpu.sync_copy(table_hbm, table_v)
        pltpu.sync_copy(wt_hbm, wt_v)
        w = [[wt_v[a, pl.ds(LANES * j, LANES)] for j in range(NJ)]
             for a in range(ATTR_DIM)]

        def wait_out(p):
            for bb in range(PB):
                pltpu.make_async_copy(
                    outb[p].at[pl.ds(bb * CN, CN), :],
                    out_hbm.at[pl.ds(0, CN)], sout[p]).wait()

        def chunk_body(c, carry):
            n0 = c * CN
            pltpu.sync_copy(ids_hbm.at[pl.ds(n0, CN), pl.ds(b0w, PANEL)],
                            ids_v.at[pl.ds(0, CN), :])
            pltpu.sync_copy(
                attr_hbm.at[pl.ds(n0, CN), :, pl.ds(b0w, PANEL)],
                attr_v.at[pl.ds(0, CN), :, :])

            def piece_pair(pp, carry2):
                for p in range(2):
                    pb = 2 * pp + p
                    bcol = PB * pb

                    @pl.when(c * n_pieces + pb >= 2)
                    def _():
                        wait_out(p)

                    out_v = outb[p]

                    @plsc.parallel_loop(0, CN)
                    def _(n):
                        idsg = ids_v[n, pl.ds(bcol, LANES)]
                        avg = [attr_v[n, a, pl.ds(bcol, LANES)]
                               for a in range(ATTR_DIM)]
                        for bb in range(PB):
                            pid = idsg[bb]
                            a0 = avg[0][bb]
                            a1 = avg[1][bb]
                            a2 = avg[2][bb]
                            a3 = avg[3][bb]
                            for j in range(NJ):
                                e = table_v[pid, pl.ds(LANES * j, LANES)]
                                r = e + a0 * w[0][j] + a1 * w[1][j] \
                                    + a2 * w[2][j] + a3 * w[3][j]
                                out_v[bb * CN + n, pl.ds(LANES * j, LANES)] = r

                    for bb in range(PB):
                        row0 = (b0w + bcol + bb) * n_n + n0
                        pltpu.async_copy(out_v.at[pl.ds(bb * CN, CN), :],
                                         out_hbm.at[pl.ds(row0, CN)], sout[p])
                return carry2

            lax.fori_loop(0, n_pieces // 2, piece_pair, 0)
            return carry

        lax.fori_loop(0, n_chunks, chunk_body, 0)
        wait_out(0)
        wait_out(1)

    return k(ids_t, attr_t, table, wt)


def _tc_kernel(ids_ref, attr_ref, tabp_ref, wt_ref, alias_ref, out_ref):
    t = TC_BB * TC_BN
    idsb = jnp.transpose(ids_ref[...].astype(jnp.float32))
    iota = lax.broadcasted_iota(jnp.int32,
                                (TC_BB, TC_BN, D_MODEL), 2).astype(jnp.float32)
    oh = (idsb[:, :, None] == iota).astype(jnp.float32)
    base = jnp.dot(oh.reshape(t, D_MODEL), tabp_ref[...],
                   preferred_element_type=jnp.float32)
    acc = base.reshape(TC_BB, TC_BN, D_MODEL)
    for a in range(ATTR_DIM):
        atta = jnp.transpose(attr_ref[:, a, :])
        acc = acc + atta[:, :, None] * wt_ref[a, :][None, None, :]
    out_ref[...] = acc


def _tc_fill(out3, ids_t, attr_t, tab_pad, wt, n_sc):
    N, B = ids_t.shape
    joff = n_sc // TC_BN
    grid = (B // TC_BB, (N - n_sc) // TC_BN)
    return pl.pallas_call(
        _tc_kernel,
        grid=grid,
        in_specs=[
            pl.BlockSpec((TC_BN, TC_BB), lambda i, j: (j + joff, i)),
            pl.BlockSpec((TC_BN, ATTR_DIM, TC_BB),
                         lambda i, j: (j + joff, 0, i)),
            pl.BlockSpec((D_MODEL, D_MODEL), lambda i, j: (0, 0)),
            pl.BlockSpec((ATTR_DIM, D_MODEL), lambda i, j: (0, 0)),
            pl.BlockSpec(memory_space=pl.ANY),
        ],
        out_specs=pl.BlockSpec((TC_BB, TC_BN, D_MODEL),
                               lambda i, j: (i, j + joff, 0)),
        out_shape=jax.ShapeDtypeStruct((B, N, D_MODEL), jnp.float32),
        input_output_aliases={4: 0},
    )(ids_t, attr_t, tab_pad, wt, out3)


@functools.partial(jax.jit, static_argnames=("n_b", "n_n", "n_sc"))
def _run_all(ids_t, attr_t, table, wt, n_b, n_n, n_sc):
    out = _run(ids_t, attr_t, table, wt, n_b, n_n, n_sc)
    out3 = out.reshape(n_b, n_n, D_MODEL)
    if n_sc < n_n:
        tab_pad = jnp.concatenate(
            [table, jnp.zeros((D_MODEL - NUM_PRIMITIVES, D_MODEL),
                              jnp.float32)], axis=0)
        out3 = _tc_fill(out3, ids_t, attr_t, tab_pad, wt, n_sc)
    return out3


def kernel(primitive_ids, attr_vec, emb_table, attr_W):
    B, N = primitive_ids.shape
    n_sc = (N * N_SC_FRAC_NUM // (N_SC_FRAC_DEN * CN)) * CN
    if n_sc <= 0 or (N - n_sc) % TC_BN != 0 or N % CN != 0:
        n_sc = N
    ids32 = primitive_ids.astype(jnp.int32)
    attrf = attr_vec.astype(jnp.float32)
    ids_t = ids32.T
    attr_t = jnp.transpose(attrf, (1, 2, 0))
    wt = attr_W.astype(jnp.float32).T
    out3 = _run_all(ids_t, attr_t,
                    emb_table.astype(jnp.float32), wt, B, N, n_sc)
    return out3
